# baseline (speedup 1.0000x reference)
import functools

import jax
import jax.numpy as jnp
from jax import lax
from jax.experimental import pallas as pl
from jax.experimental.pallas import tpu as pltpu
from jax.experimental.pallas import tpu_sc as plsc

T = 2048
HS = 1024
FF = 4096
E = 16
TOPK = 2
R = T * TOPK

BLK = 1024
P = R + E * BLK
NB = P // BLK
FFB = 2048
FFC = FF // FFB

CSB = 512
NCS = R // CSB
LW = 128

NC, NS = 2, 16
NW = NC * NS
TPW = T // NW
HPW = TPW // 2


def _routing_body(te_ref, pos_ref, be_ref, oh_scr, c_scr):
    te = te_ref[...]
    eio = lax.broadcasted_iota(jnp.int32, (R, LW), 1)
    oh = jnp.where((te == eio) & (eio < E), 1.0, 0.0)
    oh_scr[...] = oh

    rio = lax.broadcasted_iota(jnp.int32, (CSB, CSB), 0)
    cio = lax.broadcasted_iota(jnp.int32, (CSB, CSB), 1)
    tri = jnp.where(rio >= cio, 1.0, 0.0)

    def csum_blk(b, carry):
        seg = oh_scr[pl.ds(b * CSB, CSB), :]
        cseg = jnp.dot(tri, seg, preferred_element_type=jnp.float32) + carry
        c_scr[pl.ds(b * CSB, CSB), :] = cseg
        return cseg[CSB - 1:CSB, :]

    hist = lax.fori_loop(0, NCS, csum_blk, jnp.zeros((1, LW), jnp.float32))

    pe = jnp.floor((hist + (BLK - 1)) / BLK) * BLK
    i2 = lax.broadcasted_iota(jnp.int32, (LW, LW), 0)
    j2 = lax.broadcasted_iota(jnp.int32, (LW, LW), 1)
    slo = jnp.where(i2 < j2, 1.0, 0.0)
    offs = jnp.dot(jnp.broadcast_to(pe, (8, LW)), slo,
                   preferred_element_type=jnp.float32)[0:1, :]

    posf = jnp.sum(oh * (c_scr[...] - 1.0 + offs), axis=1, keepdims=True)
    pos_ref[...] = posf.astype(jnp.int32)

    bio = (lax.broadcasted_iota(jnp.int32, (NB, LW), 0) * BLK).astype(jnp.float32)
    jio = lax.broadcasted_iota(jnp.int32, (NB, LW), 1)
    cnt = jnp.sum(jnp.where((offs <= bio) & (jio < E), 1, 0), axis=1, keepdims=True)
    be_ref[...] = cnt - 1


def _routing(te):
    return pl.pallas_call(
        _routing_body,
        out_shape=(jax.ShapeDtypeStruct((R, 1), jnp.int32),
                   jax.ShapeDtypeStruct((NB, 1), jnp.int32)),
        scratch_shapes=[pltpu.VMEM((R, LW), jnp.float32),
                        pltpu.VMEM((R, LW), jnp.float32)],
    )(te)


def _scatter_body(x_hbm, p0_hbm, p1_hbm, routed_hbm, rows_v, i0_v, i1_v, sem):
    wid = lax.axis_index("s") * NC + lax.axis_index("c")
    t0 = wid * TPW
    pltpu.sync_copy(x_hbm.at[pl.ds(t0, TPW)], rows_v)
    pltpu.sync_copy(p0_hbm.at[pl.ds(t0, TPW)], i0_v)
    pltpu.sync_copy(p1_hbm.at[pl.ds(t0, TPW)], i1_v)
    cp0 = pltpu.async_copy(rows_v, routed_hbm.at[i0_v], sem)
    cp1 = pltpu.async_copy(rows_v, routed_hbm.at[i1_v], sem)
    cp0.wait()
    cp1.wait()


def _scatter(xt, p0, p1):
    return pl.kernel(
        _scatter_body,
        out_type=jax.ShapeDtypeStruct((P, HS), jnp.float32),
        mesh=plsc.VectorSubcoreMesh(core_axis_name="c", subcore_axis_name="s"),
        scratch_types=[pltpu.VMEM((TPW, HS), jnp.float32),
                       pltpu.VMEM((TPW,), jnp.int32),
                       pltpu.VMEM((TPW,), jnp.int32),
                       pltpu.SemaphoreType.DMA],
    )(xt, p0, p1)


def _gemm_body(be_ref, xb_ref, w1_ref, w2_ref, out_ref):
    f = pl.program_id(1)
    h = jnp.dot(xb_ref[...], w1_ref[0], preferred_element_type=jnp.float32)
    h = jax.nn.gelu(h)
    o = jnp.dot(h, w2_ref[0], preferred_element_type=jnp.float32)

    @pl.when(f == 0)
    def _():
        out_ref[...] = o

    @pl.when(f != 0)
    def _():
        out_ref[...] += o


def _gemm(be, routed, w1, w2):
    grid_spec = pltpu.PrefetchScalarGridSpec(
        num_scalar_prefetch=1,
        grid=(NB, FFC),
        in_specs=[
            pl.BlockSpec((BLK, HS), lambda b, f, be_ref: (b, 0)),
            pl.BlockSpec((1, HS, FFB), lambda b, f, be_ref: (be_ref[b], 0, f)),
            pl.BlockSpec((1, FFB, HS), lambda b, f, be_ref: (be_ref[b], f, 0)),
        ],
        out_specs=pl.BlockSpec((BLK, HS), lambda b, f, be_ref: (b, 0)),
    )
    return pl.pallas_call(
        _gemm_body,
        grid_spec=grid_spec,
        out_shape=jax.ShapeDtypeStruct((P, HS), jnp.float32),
        compiler_params=pltpu.CompilerParams(
            dimension_semantics=("arbitrary", "arbitrary")),
    )(be, routed, w1, w2)


def _combine_body(r_hbm, p0_hbm, p1_hbm, ew_hbm, out_hbm,
                  g0_v, g1_v, i0_v, i1_v, ew_v, sem):
    wid = lax.axis_index("s") * NC + lax.axis_index("c")

    def half(hf, carry):
        t0 = wid * TPW + hf * HPW
        pltpu.sync_copy(p0_hbm.at[pl.ds(t0, HPW)], i0_v)
        pltpu.sync_copy(p1_hbm.at[pl.ds(t0, HPW)], i1_v)
        pltpu.sync_copy(ew_hbm.at[pl.ds(t0 * TOPK, HPW * TOPK)],
                        ew_v.at[pl.ds(0, HPW * TOPK)])
        cg0 = pltpu.async_copy(r_hbm.at[i0_v], g0_v, sem)
        cg1 = pltpu.async_copy(r_hbm.at[i1_v], g1_v, sem)
        cg0.wait()
        cg1.wait()

        def per_token(t, c2):
            ewc = ew_v[pl.ds(TOPK * t, 16)]
            w0 = ewc[0]
            w1s = ewc[1]

            def per_chunk(c, c3):
                s = pl.ds(c * 64, 16)
                for u in range(4):
                    su = pl.ds(c * 64 + u * 16, 16)
                    g0_v[t, su] = w0 * g0_v[t, su] + w1s * g1_v[t, su]
                return c3

            return lax.fori_loop(0, HS // 64, per_chunk, c2)

        lax.fori_loop(0, HPW, per_token, 0)
        pltpu.sync_copy(g0_v, out_hbm.at[pl.ds(t0, HPW)])
        return carry

    lax.fori_loop(0, 2, half, 0)


def _combine(routed_out, p0, p1, ewf):
    return pl.kernel(
        _combine_body,
        out_type=jax.ShapeDtypeStruct((T, HS), jnp.float32),
        mesh=plsc.VectorSubcoreMesh(core_axis_name="c", subcore_axis_name="s"),
        scratch_types=[pltpu.VMEM((HPW, HS), jnp.float32),
                       pltpu.VMEM((HPW, HS), jnp.float32),
                       pltpu.VMEM((HPW,), jnp.int32),
                       pltpu.VMEM((HPW,), jnp.int32),
                       pltpu.VMEM((HPW * TOPK + 16,), jnp.float32),
                       pltpu.SemaphoreType.DMA],
    )(routed_out, p0, p1, ewf)


def kernel(x, scores, expert_weights, expert_indices, w1, w2):
    in_shape = x.shape
    xt = x.reshape(T, HS)
    te = expert_indices.reshape(R, 1).astype(jnp.int32)

    pos, be = _routing(te)
    pos2 = pos.reshape(T, TOPK)
    p0 = pos2[:, 0]
    p1 = pos2[:, 1]

    routed = _scatter(xt, p0, p1)
    routed_out = _gemm(be.reshape(NB), routed, w1, w2)
    out = _combine(routed_out, p0, p1, expert_weights.reshape(R))
    return out.reshape(in_shape)


# speedup vs baseline: 1.1394x; 1.1394x over previous
import functools

import jax
import jax.numpy as jnp
from jax import lax
from jax.experimental import pallas as pl
from jax.experimental.pallas import tpu as pltpu
from jax.experimental.pallas import tpu_sc as plsc

T = 2048
HS = 1024
FF = 4096
E = 16
TOPK = 2
R = T * TOPK

BLK = 512
P = R + E * BLK
NB = P // BLK
FFB = 1024
FFC = FF // FFB

CSB = 512
NCS = R // CSB
LW = 128

NC, NS = 2, 16
NW = NC * NS
TPW = T // NW
HPW = TPW // 2


def _routing_body(te_ref, pos_ref, be_ref, oh_scr, c_scr):
    te = te_ref[...]
    eio = lax.broadcasted_iota(jnp.int32, (R, LW), 1)
    oh = jnp.where((te == eio) & (eio < E), 1.0, 0.0)
    oh_scr[...] = oh

    rio = lax.broadcasted_iota(jnp.int32, (CSB, CSB), 0)
    cio = lax.broadcasted_iota(jnp.int32, (CSB, CSB), 1)
    tri = jnp.where(rio >= cio, 1.0, 0.0)

    def csum_blk(b, carry):
        seg = oh_scr[pl.ds(b * CSB, CSB), :]
        cseg = jnp.dot(tri, seg, preferred_element_type=jnp.float32) + carry
        c_scr[pl.ds(b * CSB, CSB), :] = cseg
        return cseg[CSB - 1:CSB, :]

    hist = lax.fori_loop(0, NCS, csum_blk, jnp.zeros((1, LW), jnp.float32))

    pe = jnp.floor((hist + (BLK - 1)) / BLK) * BLK
    i2 = lax.broadcasted_iota(jnp.int32, (LW, LW), 0)
    j2 = lax.broadcasted_iota(jnp.int32, (LW, LW), 1)
    slo = jnp.where(i2 < j2, 1.0, 0.0)
    offs = jnp.dot(jnp.broadcast_to(pe, (8, LW)), slo,
                   preferred_element_type=jnp.float32)[0:1, :]

    posf = jnp.sum(oh * (c_scr[...] - 1.0 + offs), axis=1, keepdims=True)
    pos_ref[...] = posf.astype(jnp.int32)

    bio = (lax.broadcasted_iota(jnp.int32, (NB, LW), 0) * BLK).astype(jnp.float32)
    jio = lax.broadcasted_iota(jnp.int32, (NB, LW), 1)
    cnt = jnp.sum(jnp.where((offs <= bio) & (jio < E), 1, 0), axis=1, keepdims=True)
    be_ref[...] = cnt - 1


def _routing(te):
    return pl.pallas_call(
        _routing_body,
        out_shape=(jax.ShapeDtypeStruct((R, 1), jnp.int32),
                   jax.ShapeDtypeStruct((NB, 1), jnp.int32)),
        scratch_shapes=[pltpu.VMEM((R, LW), jnp.float32),
                        pltpu.VMEM((R, LW), jnp.float32)],
    )(te)


def _scatter_body(x_hbm, p0_hbm, p1_hbm, routed_hbm, rows_v, i0_v, i1_v, sem):
    wid = lax.axis_index("s") * NC + lax.axis_index("c")
    t0 = wid * TPW
    pltpu.sync_copy(x_hbm.at[pl.ds(t0, TPW)], rows_v)
    pltpu.sync_copy(p0_hbm.at[pl.ds(t0, TPW)], i0_v)
    pltpu.sync_copy(p1_hbm.at[pl.ds(t0, TPW)], i1_v)
    cp0 = pltpu.async_copy(rows_v, routed_hbm.at[i0_v], sem)
    cp1 = pltpu.async_copy(rows_v, routed_hbm.at[i1_v], sem)
    cp0.wait()
    cp1.wait()


def _scatter(xt, p0, p1):
    return pl.kernel(
        _scatter_body,
        out_type=jax.ShapeDtypeStruct((P, HS), jnp.float32),
        mesh=plsc.VectorSubcoreMesh(core_axis_name="c", subcore_axis_name="s"),
        scratch_types=[pltpu.VMEM((TPW, HS), jnp.float32),
                       pltpu.VMEM((TPW,), jnp.int32),
                       pltpu.VMEM((TPW,), jnp.int32),
                       pltpu.SemaphoreType.DMA],
    )(xt, p0, p1)


def _gemm_body(be_ref, xb_ref, w1_ref, w2_ref, out_ref):
    f = pl.program_id(1)
    h = jnp.dot(xb_ref[...], w1_ref[0], preferred_element_type=jnp.float32)
    h = jax.nn.gelu(h)
    o = jnp.dot(h, w2_ref[0], preferred_element_type=jnp.float32)

    @pl.when(f == 0)
    def _():
        out_ref[...] = o

    @pl.when(f != 0)
    def _():
        out_ref[...] += o


def _gemm(be, routed, w1, w2):
    grid_spec = pltpu.PrefetchScalarGridSpec(
        num_scalar_prefetch=1,
        grid=(NB, FFC),
        in_specs=[
            pl.BlockSpec((BLK, HS), lambda b, f, be_ref: (b, 0)),
            pl.BlockSpec((1, HS, FFB), lambda b, f, be_ref: (be_ref[b], 0, f)),
            pl.BlockSpec((1, FFB, HS), lambda b, f, be_ref: (be_ref[b], f, 0)),
        ],
        out_specs=pl.BlockSpec((BLK, HS), lambda b, f, be_ref: (b, 0)),
    )
    return pl.pallas_call(
        _gemm_body,
        grid_spec=grid_spec,
        out_shape=jax.ShapeDtypeStruct((P, HS), jnp.float32),
        compiler_params=pltpu.CompilerParams(
            dimension_semantics=("arbitrary", "arbitrary")),
    )(be, routed, w1, w2)


def _combine_body(r_hbm, p0_hbm, p1_hbm, ew_hbm, out_hbm,
                  g0_v, g1_v, i0_v, i1_v, ew_v, sem):
    wid = lax.axis_index("s") * NC + lax.axis_index("c")

    def half(hf, carry):
        t0 = wid * TPW + hf * HPW
        pltpu.sync_copy(p0_hbm.at[pl.ds(t0, HPW)], i0_v)
        pltpu.sync_copy(p1_hbm.at[pl.ds(t0, HPW)], i1_v)
        pltpu.sync_copy(ew_hbm.at[pl.ds(t0 * TOPK, HPW * TOPK)],
                        ew_v.at[pl.ds(0, HPW * TOPK)])
        cg0 = pltpu.async_copy(r_hbm.at[i0_v], g0_v, sem)
        cg1 = pltpu.async_copy(r_hbm.at[i1_v], g1_v, sem)
        cg0.wait()
        cg1.wait()

        def per_token(t, c2):
            ewc = ew_v[pl.ds(TOPK * t, 16)]
            w0 = ewc[0]
            w1s = ewc[1]

            def per_chunk(c, c3):
                s = pl.ds(c * 64, 16)
                for u in range(4):
                    su = pl.ds(c * 64 + u * 16, 16)
                    g0_v[t, su] = w0 * g0_v[t, su] + w1s * g1_v[t, su]
                return c3

            return lax.fori_loop(0, HS // 64, per_chunk, c2)

        lax.fori_loop(0, HPW, per_token, 0)
        pltpu.sync_copy(g0_v, out_hbm.at[pl.ds(t0, HPW)])
        return carry

    lax.fori_loop(0, 2, half, 0)


def _combine(routed_out, p0, p1, ewf):
    return pl.kernel(
        _combine_body,
        out_type=jax.ShapeDtypeStruct((T, HS), jnp.float32),
        mesh=plsc.VectorSubcoreMesh(core_axis_name="c", subcore_axis_name="s"),
        scratch_types=[pltpu.VMEM((HPW, HS), jnp.float32),
                       pltpu.VMEM((HPW, HS), jnp.float32),
                       pltpu.VMEM((HPW,), jnp.int32),
                       pltpu.VMEM((HPW,), jnp.int32),
                       pltpu.VMEM((HPW * TOPK + 16,), jnp.float32),
                       pltpu.SemaphoreType.DMA],
    )(routed_out, p0, p1, ewf)


def kernel(x, scores, expert_weights, expert_indices, w1, w2):
    in_shape = x.shape
    xt = x.reshape(T, HS)
    te = expert_indices.reshape(R, 1).astype(jnp.int32)

    pos, be = _routing(te)
    pos2 = pos.reshape(T, TOPK)
    p0 = pos2[:, 0]
    p1 = pos2[:, 1]

    routed = _scatter(xt, p0, p1)
    routed_out = _gemm(be.reshape(NB), routed, w1, w2)
    out = _combine(routed_out, p0, p1, expert_weights.reshape(R))
    return out.reshape(in_shape)


# speedup vs baseline: 1.2353x; 1.0842x over previous
import functools

import jax
import jax.numpy as jnp
from jax import lax
from jax.experimental import pallas as pl
from jax.experimental.pallas import tpu as pltpu
from jax.experimental.pallas import tpu_sc as plsc

T = 2048
HS = 1024
FF = 4096
E = 16
TOPK = 2
R = T * TOPK

BLK = 512
P = R + E * BLK
NB = P // BLK
FFB = 2048
FFC = FF // FFB

CSB = 512
NCS = R // CSB
LW = 128

NC, NS = 2, 16
NW = NC * NS
TPW = T // NW
HPW = TPW // 2


def _routing_body(te_ref, pos_ref, be_ref, oh_scr, c_scr):
    te = te_ref[...]
    eio = lax.broadcasted_iota(jnp.int32, (R, LW), 1)
    oh = jnp.where((te == eio) & (eio < E), 1.0, 0.0)
    oh_scr[...] = oh

    rio = lax.broadcasted_iota(jnp.int32, (CSB, CSB), 0)
    cio = lax.broadcasted_iota(jnp.int32, (CSB, CSB), 1)
    tri = jnp.where(rio >= cio, 1.0, 0.0)

    def csum_blk(b, carry):
        seg = oh_scr[pl.ds(b * CSB, CSB), :]
        cseg = jnp.dot(tri, seg, preferred_element_type=jnp.float32) + carry
        c_scr[pl.ds(b * CSB, CSB), :] = cseg
        return cseg[CSB - 1:CSB, :]

    hist = lax.fori_loop(0, NCS, csum_blk, jnp.zeros((1, LW), jnp.float32))

    pe = jnp.floor((hist + (BLK - 1)) / BLK) * BLK
    i2 = lax.broadcasted_iota(jnp.int32, (LW, LW), 0)
    j2 = lax.broadcasted_iota(jnp.int32, (LW, LW), 1)
    slo = jnp.where(i2 < j2, 1.0, 0.0)
    offs = jnp.dot(jnp.broadcast_to(pe, (8, LW)), slo,
                   preferred_element_type=jnp.float32)[0:1, :]

    posf = jnp.sum(oh * (c_scr[...] - 1.0 + offs), axis=1, keepdims=True)
    pos_ref[...] = posf.astype(jnp.int32)

    bio = (lax.broadcasted_iota(jnp.int32, (NB, LW), 0) * BLK).astype(jnp.float32)
    jio = lax.broadcasted_iota(jnp.int32, (NB, LW), 1)
    cnt = jnp.sum(jnp.where((offs <= bio) & (jio < E), 1, 0), axis=1, keepdims=True)
    be_ref[...] = cnt - 1


def _routing(te):
    return pl.pallas_call(
        _routing_body,
        out_shape=(jax.ShapeDtypeStruct((R, 1), jnp.int32),
                   jax.ShapeDtypeStruct((NB, 1), jnp.int32)),
        scratch_shapes=[pltpu.VMEM((R, LW), jnp.float32),
                        pltpu.VMEM((R, LW), jnp.float32)],
    )(te)


def _scatter_body(x_hbm, p0_hbm, p1_hbm, routed_hbm, rows_v, i0_v, i1_v, sem):
    wid = lax.axis_index("s") * NC + lax.axis_index("c")
    t0 = wid * TPW
    pltpu.sync_copy(x_hbm.at[pl.ds(t0, TPW)], rows_v)
    pltpu.sync_copy(p0_hbm.at[pl.ds(t0, TPW)], i0_v)
    pltpu.sync_copy(p1_hbm.at[pl.ds(t0, TPW)], i1_v)
    cp0 = pltpu.async_copy(rows_v, routed_hbm.at[i0_v], sem)
    cp1 = pltpu.async_copy(rows_v, routed_hbm.at[i1_v], sem)
    cp0.wait()
    cp1.wait()


def _scatter(xt, p0, p1):
    return pl.kernel(
        _scatter_body,
        out_type=jax.ShapeDtypeStruct((P, HS), jnp.float32),
        mesh=plsc.VectorSubcoreMesh(core_axis_name="c", subcore_axis_name="s"),
        scratch_types=[pltpu.VMEM((TPW, HS), jnp.float32),
                       pltpu.VMEM((TPW,), jnp.int32),
                       pltpu.VMEM((TPW,), jnp.int32),
                       pltpu.SemaphoreType.DMA],
    )(xt, p0, p1)


def _gemm_body(be_ref, xb_ref, w1_ref, w2_ref, out_ref):
    f = pl.program_id(1)
    h = jnp.dot(xb_ref[...], w1_ref[0], preferred_element_type=jnp.float32)
    h = jax.nn.gelu(h)
    o = jnp.dot(h, w2_ref[0], preferred_element_type=jnp.float32)

    @pl.when(f == 0)
    def _():
        out_ref[...] = o

    @pl.when(f != 0)
    def _():
        out_ref[...] += o


def _gemm(be, routed, w1, w2):
    grid_spec = pltpu.PrefetchScalarGridSpec(
        num_scalar_prefetch=1,
        grid=(NB, FFC),
        in_specs=[
            pl.BlockSpec((BLK, HS), lambda b, f, be_ref: (b, 0)),
            pl.BlockSpec((1, HS, FFB), lambda b, f, be_ref: (be_ref[b], 0, f)),
            pl.BlockSpec((1, FFB, HS), lambda b, f, be_ref: (be_ref[b], f, 0)),
        ],
        out_specs=pl.BlockSpec((BLK, HS), lambda b, f, be_ref: (b, 0)),
    )
    return pl.pallas_call(
        _gemm_body,
        grid_spec=grid_spec,
        out_shape=jax.ShapeDtypeStruct((P, HS), jnp.float32),
        compiler_params=pltpu.CompilerParams(
            dimension_semantics=("arbitrary", "arbitrary")),
    )(be, routed, w1, w2)


def _combine_body(r_hbm, p0_hbm, p1_hbm, ew_hbm, out_hbm,
                  g0_v, g1_v, i0_v, i1_v, ew_v, sem):
    wid = lax.axis_index("s") * NC + lax.axis_index("c")

    def half(hf, carry):
        t0 = wid * TPW + hf * HPW
        pltpu.sync_copy(p0_hbm.at[pl.ds(t0, HPW)], i0_v)
        pltpu.sync_copy(p1_hbm.at[pl.ds(t0, HPW)], i1_v)
        pltpu.sync_copy(ew_hbm.at[pl.ds(t0 * TOPK, HPW * TOPK)],
                        ew_v.at[pl.ds(0, HPW * TOPK)])
        cg0 = pltpu.async_copy(r_hbm.at[i0_v], g0_v, sem)
        cg1 = pltpu.async_copy(r_hbm.at[i1_v], g1_v, sem)
        cg0.wait()
        cg1.wait()

        def per_token(t, c2):
            ewc = ew_v[pl.ds(TOPK * t, 16)]
            w0 = ewc[0]
            w1s = ewc[1]

            def per_chunk(c, c3):
                s = pl.ds(c * 64, 16)
                for u in range(4):
                    su = pl.ds(c * 64 + u * 16, 16)
                    g0_v[t, su] = w0 * g0_v[t, su] + w1s * g1_v[t, su]
                return c3

            return lax.fori_loop(0, HS // 64, per_chunk, c2)

        lax.fori_loop(0, HPW, per_token, 0)
        pltpu.sync_copy(g0_v, out_hbm.at[pl.ds(t0, HPW)])
        return carry

    lax.fori_loop(0, 2, half, 0)


def _combine(routed_out, p0, p1, ewf):
    return pl.kernel(
        _combine_body,
        out_type=jax.ShapeDtypeStruct((T, HS), jnp.float32),
        mesh=plsc.VectorSubcoreMesh(core_axis_name="c", subcore_axis_name="s"),
        scratch_types=[pltpu.VMEM((HPW, HS), jnp.float32),
                       pltpu.VMEM((HPW, HS), jnp.float32),
                       pltpu.VMEM((HPW,), jnp.int32),
                       pltpu.VMEM((HPW,), jnp.int32),
                       pltpu.VMEM((HPW * TOPK + 16,), jnp.float32),
                       pltpu.SemaphoreType.DMA],
    )(routed_out, p0, p1, ewf)


def kernel(x, scores, expert_weights, expert_indices, w1, w2):
    in_shape = x.shape
    xt = x.reshape(T, HS)
    te = expert_indices.reshape(R, 1).astype(jnp.int32)

    pos, be = _routing(te)
    pos2 = pos.reshape(T, TOPK)
    p0 = pos2[:, 0]
    p1 = pos2[:, 1]

    routed = _scatter(xt, p0, p1)
    routed_out = _gemm(be.reshape(NB), routed, w1, w2)
    out = _combine(routed_out, p0, p1, expert_weights.reshape(R))
    return out.reshape(in_shape)


# speedup vs baseline: 1.2430x; 1.0063x over previous
import functools

import jax
import jax.numpy as jnp
from jax import lax
from jax.experimental import pallas as pl
from jax.experimental.pallas import tpu as pltpu
from jax.experimental.pallas import tpu_sc as plsc

T = 2048
HS = 1024
FF = 4096
E = 16
TOPK = 2
R = T * TOPK

BLK = 512
P = R + E * BLK
NB = P // BLK
FFB = 2048
FFC = FF // FFB

CSB = 1024
NCS = R // CSB
LW = 128

NC, NS = 2, 16
NW = NC * NS
TPW = T // NW
HPW = TPW // 2


def _routing_body(te_ref, pos_ref, be_ref, oh_scr, c_scr):
    te = te_ref[...]
    eio = lax.broadcasted_iota(jnp.int32, (R, LW), 1)
    oh = jnp.where((te == eio) & (eio < E), 1.0, 0.0)
    oh_scr[...] = oh

    rio = lax.broadcasted_iota(jnp.int32, (CSB, CSB), 0)
    cio = lax.broadcasted_iota(jnp.int32, (CSB, CSB), 1)
    tri = jnp.where(rio >= cio, 1.0, 0.0)

    def csum_blk(b, carry):
        seg = oh_scr[pl.ds(b * CSB, CSB), :]
        cseg = jnp.dot(tri, seg, preferred_element_type=jnp.float32) + carry
        c_scr[pl.ds(b * CSB, CSB), :] = cseg
        return cseg[CSB - 1:CSB, :]

    hist = lax.fori_loop(0, NCS, csum_blk, jnp.zeros((1, LW), jnp.float32))

    pe = jnp.floor((hist + (BLK - 1)) / BLK) * BLK
    i2 = lax.broadcasted_iota(jnp.int32, (LW, LW), 0)
    j2 = lax.broadcasted_iota(jnp.int32, (LW, LW), 1)
    slo = jnp.where(i2 < j2, 1.0, 0.0)
    offs = jnp.dot(jnp.broadcast_to(pe, (8, LW)), slo,
                   preferred_element_type=jnp.float32)[0:1, :]

    posf = jnp.sum(oh * (c_scr[...] - 1.0 + offs), axis=1, keepdims=True)
    pos_ref[...] = posf.astype(jnp.int32)

    bio = (lax.broadcasted_iota(jnp.int32, (NB, LW), 0) * BLK).astype(jnp.float32)
    jio = lax.broadcasted_iota(jnp.int32, (NB, LW), 1)
    cnt = jnp.sum(jnp.where((offs <= bio) & (jio < E), 1, 0), axis=1, keepdims=True)
    be_ref[...] = cnt - 1


def _routing(te):
    return pl.pallas_call(
        _routing_body,
        out_shape=(jax.ShapeDtypeStruct((R, 1), jnp.int32),
                   jax.ShapeDtypeStruct((NB, 1), jnp.int32)),
        scratch_shapes=[pltpu.VMEM((R, LW), jnp.float32),
                        pltpu.VMEM((R, LW), jnp.float32)],
    )(te)


def _deinterleave(pos_v, i0_v, i1_v, n):
    for c in range(n // 16):
        ii = lax.iota(jnp.int32, 16) * TOPK + c * 32
        i0_v[pl.ds(c * 16, 16)] = plsc.load_gather(pos_v, [ii])
        i1_v[pl.ds(c * 16, 16)] = plsc.load_gather(pos_v, [ii + 1])


def _scatter_body(x_hbm, pos_hbm, routed_hbm, rows_v, pos_v, i0_v, i1_v, sem):
    wid = lax.axis_index("s") * NC + lax.axis_index("c")
    t0 = wid * TPW
    pltpu.sync_copy(x_hbm.at[pl.ds(t0, TPW)], rows_v)
    pltpu.sync_copy(pos_hbm.at[pl.ds(t0 * TOPK, TPW * TOPK)], pos_v)
    _deinterleave(pos_v, i0_v, i1_v, TPW)
    cp0 = pltpu.async_copy(rows_v, routed_hbm.at[i0_v], sem)
    cp1 = pltpu.async_copy(rows_v, routed_hbm.at[i1_v], sem)
    cp0.wait()
    cp1.wait()


def _scatter(xt, pos):
    return pl.kernel(
        _scatter_body,
        out_type=jax.ShapeDtypeStruct((P, HS), jnp.float32),
        mesh=plsc.VectorSubcoreMesh(core_axis_name="c", subcore_axis_name="s"),
        compiler_params=pltpu.CompilerParams(needs_layout_passes=False),
        scratch_types=[pltpu.VMEM((TPW, HS), jnp.float32),
                       pltpu.VMEM((TPW * TOPK,), jnp.int32),
                       pltpu.VMEM((TPW,), jnp.int32),
                       pltpu.VMEM((TPW,), jnp.int32),
                       pltpu.SemaphoreType.DMA],
    )(xt, pos)


def _gemm_body(be_ref, xb_ref, w1_ref, w2_ref, out_ref):
    f = pl.program_id(1)
    h = jnp.dot(xb_ref[...], w1_ref[0], preferred_element_type=jnp.float32)
    h = jax.nn.gelu(h)
    o = jnp.dot(h, w2_ref[0], preferred_element_type=jnp.float32)

    @pl.when(f == 0)
    def _():
        out_ref[...] = o

    @pl.when(f != 0)
    def _():
        out_ref[...] += o


def _gemm(be, routed, w1, w2):
    grid_spec = pltpu.PrefetchScalarGridSpec(
        num_scalar_prefetch=1,
        grid=(NB, FFC),
        in_specs=[
            pl.BlockSpec((BLK, HS), lambda b, f, be_ref: (b, 0)),
            pl.BlockSpec((1, HS, FFB), lambda b, f, be_ref: (be_ref[b], 0, f)),
            pl.BlockSpec((1, FFB, HS), lambda b, f, be_ref: (be_ref[b], f, 0)),
        ],
        out_specs=pl.BlockSpec((BLK, HS), lambda b, f, be_ref: (b, 0)),
    )
    return pl.pallas_call(
        _gemm_body,
        grid_spec=grid_spec,
        out_shape=jax.ShapeDtypeStruct((P, HS), jnp.float32),
        compiler_params=pltpu.CompilerParams(
            dimension_semantics=("arbitrary", "arbitrary")),
    )(be, routed, w1, w2)


def _combine_body(r_hbm, pos_hbm, ew_hbm, out_hbm,
                  g0_v, g1_v, pos_v, i0_v, i1_v, ew_v, sem):
    wid = lax.axis_index("s") * NC + lax.axis_index("c")

    def half(hf, carry):
        t0 = wid * TPW + hf * HPW
        pltpu.sync_copy(pos_hbm.at[pl.ds(t0 * TOPK, HPW * TOPK)], pos_v)
        _deinterleave(pos_v, i0_v, i1_v, HPW)
        pltpu.sync_copy(ew_hbm.at[pl.ds(t0 * TOPK, HPW * TOPK)],
                        ew_v.at[pl.ds(0, HPW * TOPK)])
        cg0 = pltpu.async_copy(r_hbm.at[i0_v], g0_v, sem)
        cg1 = pltpu.async_copy(r_hbm.at[i1_v], g1_v, sem)
        cg0.wait()
        cg1.wait()

        def per_token(t, c2):
            ewc = ew_v[pl.ds(TOPK * t, 16)]
            w0 = ewc[0]
            w1s = ewc[1]

            def per_chunk(c, c3):
                s = pl.ds(c * 64, 16)
                for u in range(4):
                    su = pl.ds(c * 64 + u * 16, 16)
                    g0_v[t, su] = w0 * g0_v[t, su] + w1s * g1_v[t, su]
                return c3

            return lax.fori_loop(0, HS // 64, per_chunk, c2)

        lax.fori_loop(0, HPW, per_token, 0)
        pltpu.sync_copy(g0_v, out_hbm.at[pl.ds(t0, HPW)])
        return carry

    lax.fori_loop(0, 2, half, 0)


def _combine(routed_out, pos, ewf):
    return pl.kernel(
        _combine_body,
        out_type=jax.ShapeDtypeStruct((T, HS), jnp.float32),
        mesh=plsc.VectorSubcoreMesh(core_axis_name="c", subcore_axis_name="s"),
        compiler_params=pltpu.CompilerParams(needs_layout_passes=False),
        scratch_types=[pltpu.VMEM((HPW, HS), jnp.float32),
                       pltpu.VMEM((HPW, HS), jnp.float32),
                       pltpu.VMEM((HPW * TOPK,), jnp.int32),
                       pltpu.VMEM((HPW,), jnp.int32),
                       pltpu.VMEM((HPW,), jnp.int32),
                       pltpu.VMEM((HPW * TOPK + 16,), jnp.float32),
                       pltpu.SemaphoreType.DMA],
    )(routed_out, pos, ewf)


def kernel(x, scores, expert_weights, expert_indices, w1, w2):
    in_shape = x.shape
    xt = x.reshape(T, HS)
    te = expert_indices.reshape(R, 1).astype(jnp.int32)

    pos, be = _routing(te)
    posf = pos.reshape(R)

    routed = _scatter(xt, posf)
    routed_out = _gemm(be.reshape(NB), routed, w1, w2)
    out = _combine(routed_out, posf, expert_weights.reshape(R))
    return out.reshape(in_shape)


# speedup vs baseline: 1.3061x; 1.0507x over previous
import functools

import jax
import jax.numpy as jnp
from jax import lax
from jax.experimental import pallas as pl
from jax.experimental.pallas import tpu as pltpu
from jax.experimental.pallas import tpu_sc as plsc

T = 2048
HS = 1024
FF = 4096
E = 16
TOPK = 2
R = T * TOPK

BLK = 512
P = R + E * BLK
NB = P // BLK
FFB = 2048
FFC = FF // FFB

CSB = 1024
NCS = R // CSB
LW = 128

NC, NS = 2, 16
NW = NC * NS
TPW = T // NW
HPW = TPW // 2


def _routing_body(te_ref, pos_ref, be_ref, oh_scr, c_scr):
    te = te_ref[...]
    eio = lax.broadcasted_iota(jnp.int32, (R, LW), 1)
    oh = jnp.where((te == eio) & (eio < E), 1.0, 0.0)
    oh_scr[...] = oh

    rio = lax.broadcasted_iota(jnp.int32, (CSB, CSB), 0)
    cio = lax.broadcasted_iota(jnp.int32, (CSB, CSB), 1)
    tri = jnp.where(rio >= cio, 1.0, 0.0)

    def csum_blk(b, carry):
        seg = oh_scr[pl.ds(b * CSB, CSB), :]
        cseg = jnp.dot(tri, seg, preferred_element_type=jnp.float32) + carry
        c_scr[pl.ds(b * CSB, CSB), :] = cseg
        return cseg[CSB - 1:CSB, :]

    hist = lax.fori_loop(0, NCS, csum_blk, jnp.zeros((1, LW), jnp.float32))

    pe = jnp.floor((hist + (BLK - 1)) / BLK) * BLK
    i2 = lax.broadcasted_iota(jnp.int32, (LW, LW), 0)
    j2 = lax.broadcasted_iota(jnp.int32, (LW, LW), 1)
    slo = jnp.where(i2 < j2, 1.0, 0.0)
    offs = jnp.dot(jnp.broadcast_to(pe, (8, LW)), slo,
                   preferred_element_type=jnp.float32)[0:1, :]

    posf = jnp.sum(oh * (c_scr[...] - 1.0 + offs), axis=1, keepdims=True)
    pos_ref[...] = posf.astype(jnp.int32)

    bio = (lax.broadcasted_iota(jnp.int32, (NB, LW), 0) * BLK).astype(jnp.float32)
    jio = lax.broadcasted_iota(jnp.int32, (NB, LW), 1)
    cnt = jnp.sum(jnp.where((offs <= bio) & (jio < E), 1, 0), axis=1, keepdims=True)
    be_ref[...] = cnt - 1


def _routing(te):
    return pl.pallas_call(
        _routing_body,
        out_shape=(jax.ShapeDtypeStruct((R, 1), jnp.int32),
                   jax.ShapeDtypeStruct((NB, 1), jnp.int32)),
        scratch_shapes=[pltpu.VMEM((R, LW), jnp.float32),
                        pltpu.VMEM((R, LW), jnp.float32)],
    )(te)


def _deinterleave(pos_v, i0_v, i1_v, n):
    for c in range(n // 16):
        ii = lax.iota(jnp.int32, 16) * TOPK + c * 32
        i0_v[pl.ds(c * 16, 16)] = plsc.load_gather(pos_v, [ii])
        i1_v[pl.ds(c * 16, 16)] = plsc.load_gather(pos_v, [ii + 1])


def _scatter_body(x_hbm, pos_hbm, routed_hbm, rows_v, pos_v, i0_v, i1_v, sem):
    wid = lax.axis_index("s") * NC + lax.axis_index("c")
    t0 = wid * TPW
    pltpu.sync_copy(x_hbm.at[pl.ds(t0, TPW)], rows_v)
    pltpu.sync_copy(pos_hbm.at[pl.ds(t0 * TOPK, TPW * TOPK)], pos_v)
    _deinterleave(pos_v, i0_v, i1_v, TPW)
    cp0 = pltpu.async_copy(rows_v, routed_hbm.at[i0_v], sem)
    cp1 = pltpu.async_copy(rows_v, routed_hbm.at[i1_v], sem)
    cp0.wait()
    cp1.wait()


def _scatter(xt, pos):
    return pl.kernel(
        _scatter_body,
        out_type=jax.ShapeDtypeStruct((P, HS), jnp.float32),
        mesh=plsc.VectorSubcoreMesh(core_axis_name="c", subcore_axis_name="s"),
        compiler_params=pltpu.CompilerParams(needs_layout_passes=False),
        scratch_types=[pltpu.VMEM((TPW, HS), jnp.float32),
                       pltpu.VMEM((TPW * TOPK,), jnp.int32),
                       pltpu.VMEM((TPW,), jnp.int32),
                       pltpu.VMEM((TPW,), jnp.int32),
                       pltpu.SemaphoreType.DMA],
    )(xt, pos)


def _gemm_body(be_ref, xb_ref, w1_ref, w2_ref, out_ref):
    f = pl.program_id(1)
    h = jnp.dot(xb_ref[...], w1_ref[0], preferred_element_type=jnp.float32)
    h = jax.nn.gelu(h)
    o = jnp.dot(h, w2_ref[0], preferred_element_type=jnp.float32)

    @pl.when(f == 0)
    def _():
        out_ref[...] = o

    @pl.when(f != 0)
    def _():
        out_ref[...] += o


def _gemm(be, routed, w1, w2):
    grid_spec = pltpu.PrefetchScalarGridSpec(
        num_scalar_prefetch=1,
        grid=(NB, FFC),
        in_specs=[
            pl.BlockSpec((BLK, HS), lambda b, f, be_ref: (b, 0)),
            pl.BlockSpec((1, HS, FFB), lambda b, f, be_ref: (be_ref[b], 0, f)),
            pl.BlockSpec((1, FFB, HS), lambda b, f, be_ref: (be_ref[b], f, 0)),
        ],
        out_specs=pl.BlockSpec((BLK, HS), lambda b, f, be_ref: (b, 0)),
    )
    return pl.pallas_call(
        _gemm_body,
        grid_spec=grid_spec,
        out_shape=jax.ShapeDtypeStruct((P, HS), jnp.float32),
        compiler_params=pltpu.CompilerParams(
            dimension_semantics=("arbitrary", "arbitrary")),
    )(be, routed, w1, w2)


def _combine_body(r_hbm, pos_hbm, ew_hbm, out_hbm,
                  g0_v, g1_v, pos_v, i0_v, i1_v, ew_v, sem):
    wid = lax.axis_index("s") * NC + lax.axis_index("c")

    def half(hf, carry):
        t0 = wid * TPW + hf * HPW
        pltpu.sync_copy(pos_hbm.at[pl.ds(t0 * TOPK, HPW * TOPK)], pos_v)
        _deinterleave(pos_v, i0_v, i1_v, HPW)
        pltpu.sync_copy(ew_hbm.at[pl.ds(t0 * TOPK, HPW * TOPK)],
                        ew_v.at[pl.ds(0, HPW * TOPK)])
        cg0 = pltpu.async_copy(r_hbm.at[i0_v], g0_v, sem)
        cg1 = pltpu.async_copy(r_hbm.at[i1_v], g1_v, sem)
        cg0.wait()
        cg1.wait()

        def per_token(t, c2):
            ewc = ew_v[pl.ds(TOPK * t, 16)]
            w0 = ewc[0]
            w1s = ewc[1]

            for u in range(HS // 16):
                su = pl.ds(u * 16, 16)
                g0_v[t, su] = w0 * g0_v[t, su] + w1s * g1_v[t, su]
            return c2

        lax.fori_loop(0, HPW, per_token, 0)
        pltpu.sync_copy(g0_v, out_hbm.at[pl.ds(t0, HPW)])
        return carry

    lax.fori_loop(0, 2, half, 0)


def _combine(routed_out, pos, ewf):
    return pl.kernel(
        _combine_body,
        out_type=jax.ShapeDtypeStruct((T, HS), jnp.float32),
        mesh=plsc.VectorSubcoreMesh(core_axis_name="c", subcore_axis_name="s"),
        compiler_params=pltpu.CompilerParams(needs_layout_passes=False),
        scratch_types=[pltpu.VMEM((HPW, HS), jnp.float32),
                       pltpu.VMEM((HPW, HS), jnp.float32),
                       pltpu.VMEM((HPW * TOPK,), jnp.int32),
                       pltpu.VMEM((HPW,), jnp.int32),
                       pltpu.VMEM((HPW,), jnp.int32),
                       pltpu.VMEM((HPW * TOPK + 16,), jnp.float32),
                       pltpu.SemaphoreType.DMA],
    )(routed_out, pos, ewf)


def kernel(x, scores, expert_weights, expert_indices, w1, w2):
    in_shape = x.shape
    xt = x.reshape(T, HS)
    te = expert_indices.reshape(R, 1).astype(jnp.int32)

    pos, be = _routing(te)
    posf = pos.reshape(R)

    routed = _scatter(xt, posf)
    routed_out = _gemm(be.reshape(NB), routed, w1, w2)
    out = _combine(routed_out, posf, expert_weights.reshape(R))
    return out.reshape(in_shape)
